# baseline (speedup 1.0000x reference)
import functools

import jax
import jax.numpy as jnp
from jax import lax
from jax.experimental import pallas as pl
from jax.experimental.pallas import tpu as pltpu
from jax.experimental.pallas import tpu_sc as plsc

_NUM_BINS = 15
_NSLOTS = 16
_LANES = 16
_ACC_WORDS = 3 * _NSLOTS * _LANES

_NC = 2
_NS = 16
_NW = _NC * _NS

_CHUNK = 16384
_UNROLL = 4


def _sc_partials(conf, acc):
    n = conf.shape[0]
    per_w = n // _NW
    n_chunks = per_w // _CHUNK
    vec_steps = _CHUNK // (_LANES * _UNROLL)

    mesh = plsc.VectorSubcoreMesh(core_axis_name="c", subcore_axis_name="s")

    @functools.partial(
        pl.kernel,
        mesh=mesh,
        out_type=jax.ShapeDtypeStruct((_NW, _ACC_WORDS), jnp.float32),
        scratch_types=[
            pltpu.VMEM((_CHUNK,), jnp.float32),
            pltpu.VMEM((_CHUNK,), jnp.int32),
            pltpu.VMEM((_ACC_WORDS,), jnp.float32),
        ],
        compiler_params=pltpu.CompilerParams(needs_layout_passes=False),
    )
    def k(conf_hbm, acc_hbm, out_hbm, conf_v, acc_v, accum_v):
        wid = lax.axis_index("s") * _NC + lax.axis_index("c")
        base = wid * per_w
        zeros = jnp.zeros((_LANES,), jnp.float32)
        for i in range(_ACC_WORDS // _LANES):
            accum_v[pl.ds(i * _LANES, _LANES)] = zeros
        lane = lax.iota(jnp.int32, _LANES)
        ones = jnp.ones((_LANES,), jnp.float32)

        def chunk_body(ci, carry):
            off = base + ci * _CHUNK
            pltpu.sync_copy(conf_hbm.at[pl.ds(off, _CHUNK)], conf_v)
            pltpu.sync_copy(acc_hbm.at[pl.ds(off, _CHUNK)], acc_v)

            def vec_body(vi, c2):
                s0 = vi * (_LANES * _UNROLL)
                for u in range(_UNROLL):
                    s = s0 + u * _LANES
                    c = conf_v[pl.ds(s, _LANES)]
                    a = acc_v[pl.ds(s, _LANES)]
                    slot = jnp.minimum((c * 15.0).astype(jnp.int32) + 1, 15)
                    slot = jnp.where(c > 0.0, slot, 0)
                    idx = slot * _LANES + lane
                    plsc.addupdate_scatter(accum_v, [idx], ones)
                    plsc.addupdate_scatter(accum_v, [idx + 256], c)
                    plsc.addupdate_scatter(
                        accum_v, [idx + 512], a.astype(jnp.float32))
                return c2

            lax.fori_loop(0, vec_steps, vec_body, 0)
            return carry

        lax.fori_loop(0, n_chunks, chunk_body, 0)
        pltpu.sync_copy(accum_v, out_hbm.at[wid])

    return k(conf, acc)


def _finalize(partials, n):
    inv_n = 1.0 / float(n)

    def body(p_ref, o_ref):
        p = p_ref[...]
        t = jnp.sum(p, axis=2)
        cnt = t[0:1, :]
        cf = t[1:2, :]
        ac = t[2:3, :]
        safe = jnp.maximum(cnt, 1.0)
        term = jnp.abs(cf / safe - ac / safe) * (cnt * inv_n)
        slot = lax.broadcasted_iota(jnp.int32, (1, _NSLOTS), 1)
        term = jnp.where((slot >= 1) & (cnt > 0.0), term, 0.0)
        o_ref[0, 0] = jnp.sum(term)

    return pl.pallas_call(
        body,
        out_shape=jax.ShapeDtypeStruct((1, 1), jnp.float32),
        out_specs=pl.BlockSpec(memory_space=pltpu.SMEM),
    )(partials)


def kernel(confidences, accuracies):
    parts = _sc_partials(confidences, accuracies)
    parts = (
        parts.reshape(_NW, 3, _NSLOTS, _LANES)
        .transpose(1, 2, 0, 3)
        .reshape(3, _NSLOTS, _NW * _LANES)
    )
    return _finalize(parts, confidences.shape[0])[0, 0]


# speedup vs baseline: 2.7858x; 2.7858x over previous
import functools

import jax
import jax.numpy as jnp
from jax import lax
from jax.experimental import pallas as pl
from jax.experimental.pallas import tpu as pltpu
from jax.experimental.pallas import tpu_sc as plsc

_NUM_BINS = 15
_NSLOTS = 16
_LANES = 16
_ACC_WORDS = 3 * _NSLOTS * _LANES

_NC = 2
_NS = 16
_NW = _NC * _NS

_CHUNK = 16384
_UNROLL = 8


def _sc_partials(conf, acc):
    n = conf.shape[0]
    per_w = n // _NW
    n_chunks = per_w // _CHUNK
    vec_steps = _CHUNK // (_LANES * _UNROLL)

    mesh = plsc.VectorSubcoreMesh(core_axis_name="c", subcore_axis_name="s")

    @functools.partial(
        pl.kernel,
        mesh=mesh,
        out_type=jax.ShapeDtypeStruct((_NW, _ACC_WORDS), jnp.float32),
        scratch_types=[
            pltpu.VMEM((_CHUNK,), jnp.float32),
            pltpu.VMEM((_CHUNK,), jnp.int32),
            pltpu.VMEM((_ACC_WORDS,), jnp.float32),
        ],
        compiler_params=pltpu.CompilerParams(needs_layout_passes=False),
    )
    def k(conf_hbm, acc_hbm, out_hbm, conf_v, acc_v, accum_v):
        wid = lax.axis_index("s") * _NC + lax.axis_index("c")
        base = wid * per_w
        zeros = jnp.zeros((_LANES,), jnp.float32)
        for i in range(_ACC_WORDS // _LANES):
            accum_v[pl.ds(i * _LANES, _LANES)] = zeros
        lane = lax.iota(jnp.int32, _LANES)
        ones = jnp.ones((_LANES,), jnp.float32)

        def chunk_body(ci, carry):
            off = base + ci * _CHUNK
            pltpu.sync_copy(conf_hbm.at[pl.ds(off, _CHUNK)], conf_v)
            pltpu.sync_copy(acc_hbm.at[pl.ds(off, _CHUNK)], acc_v)

            @plsc.parallel_loop(0, _CHUNK, step=_LANES, unroll=_UNROLL)
            def vec_body(s):
                c = conf_v[pl.ds(s, _LANES)]
                a = acc_v[pl.ds(s, _LANES)]
                slot = jnp.minimum((c * 15.0).astype(jnp.int32) + 1, 15)
                slot = jnp.where(c > 0.0, slot, 0)
                idx = slot * _LANES + lane
                plsc.addupdate_scatter(accum_v, [idx], ones)
                plsc.addupdate_scatter(accum_v, [idx + 256], c)
                plsc.addupdate_scatter(
                    accum_v, [idx + 512], a.astype(jnp.float32))

            return carry

        lax.fori_loop(0, n_chunks, chunk_body, 0)
        pltpu.sync_copy(accum_v, out_hbm.at[wid])

    return k(conf, acc)


def _finalize(partials, n):
    inv_n = 1.0 / float(n)

    def body(p_ref, o_ref):
        p = p_ref[...]
        t = jnp.sum(p, axis=2)
        cnt = t[0:1, :]
        cf = t[1:2, :]
        ac = t[2:3, :]
        safe = jnp.maximum(cnt, 1.0)
        term = jnp.abs(cf / safe - ac / safe) * (cnt * inv_n)
        slot = lax.broadcasted_iota(jnp.int32, (1, _NSLOTS), 1)
        term = jnp.where((slot >= 1) & (cnt > 0.0), term, 0.0)
        o_ref[0, 0] = jnp.sum(term)

    return pl.pallas_call(
        body,
        out_shape=jax.ShapeDtypeStruct((1, 1), jnp.float32),
        out_specs=pl.BlockSpec(memory_space=pltpu.SMEM),
    )(partials)


def kernel(confidences, accuracies):
    parts = _sc_partials(confidences, accuracies)
    parts = (
        parts.reshape(_NW, 3, _NSLOTS, _LANES)
        .transpose(1, 2, 0, 3)
        .reshape(3, _NSLOTS, _NW * _LANES)
    )
    return _finalize(parts, confidences.shape[0])[0, 0]
